# baseline (speedup 1.0000x reference)
import functools

import numpy as np
import jax
import jax.numpy as jnp
from jax import lax
from jax.experimental import pallas as pl
from jax.experimental.pallas import tpu as pltpu
from jax.experimental.pallas import tpu_sc as plsc

GS = 128
N = GS * GS
R = 15
K = 2 * R + 1
KP = 32
NB = 1089
NSHIFT = K * K
NPAD = NB - NSHIFT
T = 10.0
L = 16
DT_LEN = K * KP
PADW = GS + 2 * R

_dt = np.full((K, KP), 1.0e4, np.float32)
for _si in range(K):
    for _sj in range(K):
        _dt[_si, _sj] = np.sqrt((_si - R) ** 2 + (_sj - R) ** 2) / R
_DIST_T = _dt.reshape(-1)
_D_SPECIAL = np.array([0.0, np.sqrt(2.0) * (GS // 2) / R] + [1.0e4] * (L - 2),
                      np.float32)


def _wterm(d, p_rk, p_b, p_w, p_r):
    z = (d / p_r - p_rk) / p_w
    return p_b * jnp.exp(-(z * z) / 2.0)


def _sc_body(pk_hbm, wn_hbm, pk_v, wt_v, *, nb_rules, nterms, npk):
    wid = lax.axis_index("s") * 2 + lax.axis_index("c")

    @pl.when(wid == 0)
    def _():
        pltpu.sync_copy(pk_hbm, pk_v)

        spec = pk_v[pl.ds(DT_LEN, L)]
        nhdr = ((4 * nb_rules + L - 1) // L) * L
        hchunks = [pk_v[pl.ds(DT_LEN + L + c * L, L)] for c in range(nhdr // L)]
        tbase = DT_LEN + L + nhdr
        ntrm = ((3 * nb_rules * nterms + L - 1) // L) * L
        tchunks = [pk_v[pl.ds(tbase + c * L, L)] for c in range(ntrm // L)]

        def term_params(kr, lt):
            base = (kr * nterms + lt) * 3
            return (tchunks[base // L][base % L],
                    tchunks[(base + 1) // L][(base + 1) % L],
                    tchunks[(base + 2) // L][(base + 2) % L])

        for kr in range(nb_rules):
            p_r = hchunks[(kr * 4) // L][(kr * 4) % L]
            sig_spec = 1.0 / (1.0 + jnp.exp((spec - 1.0) * 10.0))
            wspec = jnp.zeros((L,), jnp.float32)
            for lt in range(nterms):
                p_rk, p_b, p_w = term_params(kr, lt)
                wspec = wspec + _wterm(spec, p_rk, p_b, p_w, p_r)
            wspec = sig_spec * wspec
            wsum_vec = jnp.zeros((L,), jnp.float32)
            wvs = []
            for c in range(DT_LEN // L):
                dvec = pk_v[pl.ds(c * L, L)]
                sig = 1.0 / (1.0 + jnp.exp((dvec - 1.0) * 10.0))
                core = jnp.zeros((L,), jnp.float32)
                for lt in range(nterms):
                    p_rk, p_b, p_w = term_params(kr, lt)
                    core = core + _wterm(dvec, p_rk, p_b, p_w, p_r)
                wv = sig * core
                wvs.append(wv)
                wsum_vec = wsum_vec + wv
            w0 = wspec[0]
            wpad = wspec[1]
            wsum = w0 + NPAD * wpad
            for l in range(L):
                wsum = wsum + wsum_vec[l]
            inv_v = 1.0 / (jnp.full((L,), wsum) * float(NB + 1))
            for c in range(DT_LEN // L):
                wt_v[pl.ds(kr * DT_LEN + c * L, L)] = wvs[c] * inv_v
            inv0 = inv_v[0]
            lanes = lax.iota(jnp.int32, L)
            sl_vec = jnp.where(lanes == 0, inv0 * w0,
                               jnp.where(lanes == 1,
                                         inv0 * (float(NPAD) * wpad), 0.0))
            wt_v[pl.ds(nb_rules * DT_LEN + kr * L, L)] = sl_vec
        pltpu.sync_copy(wt_v, wn_hbm)


def _run_sc(pk, nb_rules, nterms):
    mesh = plsc.VectorSubcoreMesh(core_axis_name="c", subcore_axis_name="s")
    body = functools.partial(_sc_body, nb_rules=nb_rules, nterms=nterms,
                             npk=pk.shape[0])
    f = pl.kernel(
        body,
        out_type=jax.ShapeDtypeStruct((nb_rules * (DT_LEN + L),), jnp.float32),
        mesh=mesh,
        compiler_params=pltpu.CompilerParams(needs_layout_passes=False),
        scratch_types=[
            pltpu.VMEM((pk.shape[0],), jnp.float32),
            pltpu.VMEM((nb_rules * (DT_LEN + L),), jnp.float32),
        ],
    )
    return f(pk)


def _tc1_body(x_ref, g_ref):
    xg = x_ref[...]
    xv = jnp.concatenate([xg[GS - R:, :], xg, xg[:R, :]], axis=0)
    xp = jnp.concatenate([xv[:, GS - R:], xv, xv[:, :R]], axis=1)
    g_ref[...] = jnp.concatenate([xp[si:si + GS, :] for si in range(K)],
                                 axis=1)


def _tc_body(x_ref, g_ref, wn_ref, sl_ref, h_ref, m_ref, s_ref, out_ref,
             *, nb_rules):
    xg = x_ref[...]
    G = g_ref[...]

    a3 = jax.lax.broadcasted_iota(jnp.int32, (K, PADW, GS), 1)
    j3 = jax.lax.broadcasted_iota(jnp.int32, (K, PADW, GS), 2)
    t3 = jax.lax.broadcasted_iota(jnp.int32, (K, PADW, GS), 0)
    masks = (a3 - j3 == t3).astype(jnp.float32).reshape(K, PADW * GS)

    delta = jnp.zeros_like(xg)
    for kr in range(nb_rules):
        kwn = wn_ref[kr * K:(kr + 1) * K, :K]
        w0n = sl_ref[kr:kr + 1, 0:1]
        padn = sl_ref[kr:kr + 1, 1:2]

        B = jnp.dot(kwn, masks, preferred_element_type=jnp.float32)
        B = B.reshape(K * PADW, GS)

        conv = jnp.dot(G, B, preferred_element_type=jnp.float32)
        pot = conv + w0n * xg + padn * xg[0:1, 0:1]

        mk = m_ref[kr:kr + 1, 0:1]
        sk = s_ref[kr:kr + 1, 0:1]
        field = jnp.exp(-(pot - mk) ** 2 / (2.0 * sk * sk) - 0.001) * 2.0 - 1.0
        delta = delta + h_ref[kr:kr + 1, 0:1] * field

    out_ref[...] = jnp.clip(xg + delta * (1.0 / T), 0.0, 1.0)


def _run_tc1(xg):
    return pl.pallas_call(
        _tc1_body,
        out_shape=jax.ShapeDtypeStruct((GS, K * PADW), jnp.float32),
    )(xg)


def _run_tc(xg, G, wn2d, sl2d, h, m, s):
    nb_rules = h.shape[0]
    body = functools.partial(_tc_body, nb_rules=nb_rules)
    return pl.pallas_call(
        body,
        out_shape=jax.ShapeDtypeStruct((GS, GS), jnp.float32),
    )(xg, G, wn2d, sl2d, h.reshape(-1, 1), m.reshape(-1, 1), s.reshape(-1, 1))


def _pad16(a):
    pad = (-a.shape[0]) % 16
    return jnp.pad(a, (0, pad)) if pad else a


@functools.partial(jax.jit, static_argnums=(3, 4))
def _run(xg, pk, hms, nb_rules, nterms):
    h, m, s = hms
    wnsl = _run_sc(pk, nb_rules, nterms)
    G = _run_tc1(xg)
    wn2d = wnsl[:nb_rules * DT_LEN].reshape(nb_rules * K, KP)
    sl2d = wnsl[nb_rules * DT_LEN:].reshape(nb_rules, L)
    return _run_tc(xg, G, wn2d, sl2d, h, m, s)


def kernel(pos, x, r, rk, b, w, h, m, s):
    nb_rules = r.shape[0]
    nterms = rk.shape[1]
    hdr = _pad16(jnp.stack([r, h, m, s], axis=1).reshape(-1))
    trm = _pad16(jnp.stack([rk, jnp.broadcast_to(b, rk.shape),
                            jnp.broadcast_to(w, rk.shape)], axis=2).reshape(-1))
    pk = jnp.concatenate([jnp.asarray(_DIST_T), jnp.asarray(_D_SPECIAL),
                          hdr, trm])
    out = _run(x.reshape(GS, GS), pk, (h, m, s), nb_rules, nterms)
    return (pos, out.reshape(N, 1))


# speedup vs baseline: 1.0351x; 1.0351x over previous
import functools

import numpy as np
import jax
import jax.numpy as jnp
from jax import lax
from jax.experimental import pallas as pl
from jax.experimental.pallas import tpu as pltpu
from jax.experimental.pallas import tpu_sc as plsc

GS = 128
N = GS * GS
R = 15
K = 2 * R + 1
KP = 32
NB = 1089
NSHIFT = K * K
NPAD = NB - NSHIFT
T = 10.0
L = 16
DT_LEN = K * KP
PADW = GS + 2 * R

_dt = np.full((K, KP), 1.0e4, np.float32)
for _si in range(K):
    for _sj in range(K):
        _dt[_si, _sj] = np.sqrt((_si - R) ** 2 + (_sj - R) ** 2) / R
_DIST_T = _dt.reshape(-1)
_D_SPECIAL = np.array([0.0, np.sqrt(2.0) * (GS // 2) / R] + [1.0e4] * (L - 2),
                      np.float32)


def _wterm(d, p_rk, p_b, p_w, p_r):
    z = (d / p_r - p_rk) / p_w
    return p_b * jnp.exp(-(z * z) / 2.0)


def _sc_body(pk_hbm, wn_hbm, pk_v, wt_v, *, nb_rules, nterms, npk):
    wid = lax.axis_index("s") * 2 + lax.axis_index("c")

    @pl.when(wid == 0)
    def _():
        pltpu.sync_copy(pk_hbm, pk_v)

        spec = pk_v[pl.ds(DT_LEN, L)]
        nhdr = ((4 * nb_rules + L - 1) // L) * L
        hchunks = [pk_v[pl.ds(DT_LEN + L + c * L, L)] for c in range(nhdr // L)]
        tbase = DT_LEN + L + nhdr
        ntrm = ((3 * nb_rules * nterms + L - 1) // L) * L
        tchunks = [pk_v[pl.ds(tbase + c * L, L)] for c in range(ntrm // L)]

        def term_params(kr, lt):
            base = (kr * nterms + lt) * 3
            return (tchunks[base // L][base % L],
                    tchunks[(base + 1) // L][(base + 1) % L],
                    tchunks[(base + 2) // L][(base + 2) % L])

        for kr in range(nb_rules):
            p_r = hchunks[(kr * 4) // L][(kr * 4) % L]
            sig_spec = 1.0 / (1.0 + jnp.exp((spec - 1.0) * 10.0))
            wspec = jnp.zeros((L,), jnp.float32)
            for lt in range(nterms):
                p_rk, p_b, p_w = term_params(kr, lt)
                wspec = wspec + _wterm(spec, p_rk, p_b, p_w, p_r)
            wspec = sig_spec * wspec
            wsum_vec = jnp.zeros((L,), jnp.float32)
            wvs = []
            for c in range(DT_LEN // L):
                dvec = pk_v[pl.ds(c * L, L)]
                sig = 1.0 / (1.0 + jnp.exp((dvec - 1.0) * 10.0))
                core = jnp.zeros((L,), jnp.float32)
                for lt in range(nterms):
                    p_rk, p_b, p_w = term_params(kr, lt)
                    core = core + _wterm(dvec, p_rk, p_b, p_w, p_r)
                wv = sig * core
                wvs.append(wv)
                wsum_vec = wsum_vec + wv
            w0 = wspec[0]
            wpad = wspec[1]
            wsum = w0 + NPAD * wpad
            for l in range(L):
                wsum = wsum + wsum_vec[l]
            inv_v = 1.0 / (jnp.full((L,), wsum) * float(NB + 1))
            for c in range(DT_LEN // L):
                wt_v[pl.ds(kr * DT_LEN + c * L, L)] = wvs[c] * inv_v
            inv0 = inv_v[0]
            lanes = lax.iota(jnp.int32, L)
            sl_vec = jnp.where(lanes == 0, inv0 * w0,
                               jnp.where(lanes == 1,
                                         inv0 * (float(NPAD) * wpad), 0.0))
            wt_v[pl.ds(nb_rules * DT_LEN + kr * L, L)] = sl_vec
        pltpu.sync_copy(wt_v, wn_hbm)


def _run_sc(pk, nb_rules, nterms):
    mesh = plsc.VectorSubcoreMesh(core_axis_name="c", subcore_axis_name="s")
    body = functools.partial(_sc_body, nb_rules=nb_rules, nterms=nterms,
                             npk=pk.shape[0])
    f = pl.kernel(
        body,
        out_type=jax.ShapeDtypeStruct((nb_rules * (DT_LEN + L),), jnp.float32),
        mesh=mesh,
        compiler_params=pltpu.CompilerParams(needs_layout_passes=False),
        scratch_types=[
            pltpu.VMEM((pk.shape[0],), jnp.float32),
            pltpu.VMEM((nb_rules * (DT_LEN + L),), jnp.float32),
        ],
    )
    return f(pk)


def _tc_body(x_ref, wn_ref, sl_ref, h_ref, m_ref, s_ref, out_ref, *, nb_rules):
    xg = x_ref[...]

    xv = jnp.concatenate([xg[GS - R:, :], xg, xg[:R, :]], axis=0)
    xp = jnp.concatenate([xv[:, GS - R:], xv, xv[:, :R]], axis=1)

    G = jnp.concatenate([xp[si:si + GS, :] for si in range(K)], axis=1)

    a3 = jax.lax.broadcasted_iota(jnp.int32, (K, PADW, GS), 1)
    j3 = jax.lax.broadcasted_iota(jnp.int32, (K, PADW, GS), 2)
    t3 = jax.lax.broadcasted_iota(jnp.int32, (K, PADW, GS), 0)
    masks = (a3 - j3 == t3).astype(jnp.float32).reshape(K, PADW * GS)

    delta = jnp.zeros_like(xg)
    for kr in range(nb_rules):
        kwn = wn_ref[kr * K:(kr + 1) * K, :K]
        w0n = sl_ref[kr:kr + 1, 0:1]
        padn = sl_ref[kr:kr + 1, 1:2]

        B = jnp.dot(kwn, masks, preferred_element_type=jnp.float32)
        B = B.reshape(K * PADW, GS)

        conv = jnp.dot(G, B, preferred_element_type=jnp.float32)
        pot = conv + w0n * xg + padn * xg[0:1, 0:1]

        mk = m_ref[kr:kr + 1, 0:1]
        sk = s_ref[kr:kr + 1, 0:1]
        field = jnp.exp(-(pot - mk) ** 2 / (2.0 * sk * sk) - 0.001) * 2.0 - 1.0
        delta = delta + h_ref[kr:kr + 1, 0:1] * field

    out_ref[...] = jnp.clip(xg + delta * (1.0 / T), 0.0, 1.0)


def _run_tc(xg, wn2d, sl2d, h, m, s):
    nb_rules = h.shape[0]
    body = functools.partial(_tc_body, nb_rules=nb_rules)
    return pl.pallas_call(
        body,
        out_shape=jax.ShapeDtypeStruct((GS, GS), jnp.float32),
    )(xg, wn2d, sl2d, h.reshape(-1, 1), m.reshape(-1, 1), s.reshape(-1, 1))


def _pad16(a):
    pad = (-a.shape[0]) % 16
    return jnp.pad(a, (0, pad)) if pad else a


@functools.partial(jax.jit, static_argnums=(3, 4))
def _run(xg, pk, hms, nb_rules, nterms):
    h, m, s = hms
    wnsl = _run_sc(pk, nb_rules, nterms)
    wn2d = wnsl[:nb_rules * DT_LEN].reshape(nb_rules * K, KP)
    sl2d = wnsl[nb_rules * DT_LEN:].reshape(nb_rules, L)
    return _run_tc(xg, wn2d, sl2d, h, m, s)


def kernel(pos, x, r, rk, b, w, h, m, s):
    nb_rules = r.shape[0]
    nterms = rk.shape[1]
    hdr = _pad16(jnp.stack([r, h, m, s], axis=1).reshape(-1))
    trm = _pad16(jnp.stack([rk, jnp.broadcast_to(b, rk.shape),
                            jnp.broadcast_to(w, rk.shape)], axis=2).reshape(-1))
    pk = jnp.concatenate([jnp.asarray(_DIST_T), jnp.asarray(_D_SPECIAL),
                          hdr, trm])
    out = _run(x.reshape(GS, GS), pk, (h, m, s), nb_rules, nterms)
    return (pos, out.reshape(N, 1))


# speedup vs baseline: 1.0796x; 1.0430x over previous
import functools

import numpy as np
import jax
import jax.numpy as jnp
from jax import lax
from jax.experimental import pallas as pl
from jax.experimental.pallas import tpu as pltpu
from jax.experimental.pallas import tpu_sc as plsc

GS = 128
N = GS * GS
R = 15
K = 2 * R + 1
KP = 32
NB = 1089
NSHIFT = K * K
NPAD = NB - NSHIFT
T = 10.0
L = 16
DT_LEN = K * KP
PADW = GS + 2 * R

_dt = np.full((K, KP), 1.0e4, np.float32)
for _si in range(K):
    for _sj in range(K):
        _dt[_si, _sj] = np.sqrt((_si - R) ** 2 + (_sj - R) ** 2) / R
_DIST_T = _dt.reshape(-1)
_D_SPECIAL = np.array([0.0, np.sqrt(2.0) * (GS // 2) / R] + [1.0e4] * (L - 2),
                      np.float32)


def _wterm(d, p_rk, p_b, p_w, p_r):
    z = (d / p_r - p_rk) / p_w
    return p_b * jnp.exp(-(z * z) / 2.0)


def _sc_body(pk_hbm, wn_hbm, pk_v, wt_v, *, nb_rules, nterms, npk):
    wid = lax.axis_index("s") * 2 + lax.axis_index("c")

    @pl.when(wid == 0)
    def _():
        pltpu.sync_copy(pk_hbm, pk_v)

        spec = pk_v[pl.ds(DT_LEN, L)]
        nhdr = ((4 * nb_rules + L - 1) // L) * L
        hchunks = [pk_v[pl.ds(DT_LEN + L + c * L, L)] for c in range(nhdr // L)]
        tbase = DT_LEN + L + nhdr
        ntrm = ((3 * nb_rules * nterms + L - 1) // L) * L
        tchunks = [pk_v[pl.ds(tbase + c * L, L)] for c in range(ntrm // L)]

        def term_params(kr, lt):
            base = (kr * nterms + lt) * 3
            return (tchunks[base // L][base % L],
                    tchunks[(base + 1) // L][(base + 1) % L],
                    tchunks[(base + 2) // L][(base + 2) % L])

        for kr in range(nb_rules):
            p_r = hchunks[(kr * 4) // L][(kr * 4) % L]
            sig_spec = 1.0 / (1.0 + jnp.exp((spec - 1.0) * 10.0))
            wspec = jnp.zeros((L,), jnp.float32)
            for lt in range(nterms):
                p_rk, p_b, p_w = term_params(kr, lt)
                wspec = wspec + _wterm(spec, p_rk, p_b, p_w, p_r)
            wspec = sig_spec * wspec
            wsum_vec = jnp.zeros((L,), jnp.float32)
            wvs = []
            for c in range(DT_LEN // L):
                dvec = pk_v[pl.ds(c * L, L)]
                sig = 1.0 / (1.0 + jnp.exp((dvec - 1.0) * 10.0))
                core = jnp.zeros((L,), jnp.float32)
                for lt in range(nterms):
                    p_rk, p_b, p_w = term_params(kr, lt)
                    core = core + _wterm(dvec, p_rk, p_b, p_w, p_r)
                wv = sig * core
                wvs.append(wv)
                wsum_vec = wsum_vec + wv
            w0 = wspec[0]
            wpad = wspec[1]
            wsum = w0 + NPAD * wpad
            for l in range(L):
                wsum = wsum + wsum_vec[l]
            inv_v = 1.0 / (jnp.full((L,), wsum) * float(NB + 1))
            for c in range(DT_LEN // L):
                wt_v[pl.ds(kr * DT_LEN + c * L, L)] = wvs[c] * inv_v
            inv0 = inv_v[0]
            lanes = lax.iota(jnp.int32, L)
            sl_vec = jnp.where(lanes == 0, inv0 * w0,
                               jnp.where(lanes == 1,
                                         inv0 * (float(NPAD) * wpad), 0.0))
            wt_v[pl.ds(nb_rules * DT_LEN + kr * L, L)] = sl_vec
        pltpu.sync_copy(wt_v, wn_hbm)


def _run_sc(pk, nb_rules, nterms):
    mesh = plsc.VectorSubcoreMesh(core_axis_name="c", subcore_axis_name="s",
                                  num_cores=1)
    body = functools.partial(_sc_body, nb_rules=nb_rules, nterms=nterms,
                             npk=pk.shape[0])
    f = pl.kernel(
        body,
        out_type=jax.ShapeDtypeStruct((nb_rules * (DT_LEN + L),), jnp.float32),
        mesh=mesh,
        compiler_params=pltpu.CompilerParams(needs_layout_passes=False),
        scratch_types=[
            pltpu.VMEM((pk.shape[0],), jnp.float32),
            pltpu.VMEM((nb_rules * (DT_LEN + L),), jnp.float32),
        ],
    )
    return f(pk)


def _tc_body(x_ref, wn_ref, sl_ref, h_ref, m_ref, s_ref, out_ref, *, nb_rules):
    xg = x_ref[...]

    xv = jnp.concatenate([xg[GS - R:, :], xg, xg[:R, :]], axis=0)
    xp = jnp.concatenate([xv[:, GS - R:], xv, xv[:, :R]], axis=1)

    G = jnp.concatenate([xp[si:si + GS, :] for si in range(K)], axis=1)

    a3 = jax.lax.broadcasted_iota(jnp.int32, (K, PADW, GS), 1)
    j3 = jax.lax.broadcasted_iota(jnp.int32, (K, PADW, GS), 2)
    t3 = jax.lax.broadcasted_iota(jnp.int32, (K, PADW, GS), 0)
    masks = (a3 - j3 == t3).astype(jnp.float32).reshape(K, PADW * GS)

    delta = jnp.zeros_like(xg)
    for kr in range(nb_rules):
        kwn = wn_ref[kr * K:(kr + 1) * K, :K]
        w0n = sl_ref[kr:kr + 1, 0:1]
        padn = sl_ref[kr:kr + 1, 1:2]

        B = jnp.dot(kwn, masks, preferred_element_type=jnp.float32)
        B = B.reshape(K * PADW, GS)

        conv = jnp.dot(G, B, preferred_element_type=jnp.float32)
        pot = conv + w0n * xg + padn * xg[0:1, 0:1]

        mk = m_ref[kr:kr + 1, 0:1]
        sk = s_ref[kr:kr + 1, 0:1]
        field = jnp.exp(-(pot - mk) ** 2 / (2.0 * sk * sk) - 0.001) * 2.0 - 1.0
        delta = delta + h_ref[kr:kr + 1, 0:1] * field

    out_ref[...] = jnp.clip(xg + delta * (1.0 / T), 0.0, 1.0)


def _run_tc(xg, wn2d, sl2d, h, m, s):
    nb_rules = h.shape[0]
    body = functools.partial(_tc_body, nb_rules=nb_rules)
    return pl.pallas_call(
        body,
        out_shape=jax.ShapeDtypeStruct((GS, GS), jnp.float32),
    )(xg, wn2d, sl2d, h.reshape(-1, 1), m.reshape(-1, 1), s.reshape(-1, 1))


def _pad16(a):
    pad = (-a.shape[0]) % 16
    return jnp.pad(a, (0, pad)) if pad else a


@functools.partial(jax.jit, static_argnums=(3, 4))
def _run(xg, pk, hms, nb_rules, nterms):
    h, m, s = hms
    wnsl = _run_sc(pk, nb_rules, nterms)
    wn2d = wnsl[:nb_rules * DT_LEN].reshape(nb_rules * K, KP)
    sl2d = wnsl[nb_rules * DT_LEN:].reshape(nb_rules, L)
    return _run_tc(xg, wn2d, sl2d, h, m, s)


def kernel(pos, x, r, rk, b, w, h, m, s):
    nb_rules = r.shape[0]
    nterms = rk.shape[1]
    hdr = _pad16(jnp.stack([r, h, m, s], axis=1).reshape(-1))
    trm = _pad16(jnp.stack([rk, jnp.broadcast_to(b, rk.shape),
                            jnp.broadcast_to(w, rk.shape)], axis=2).reshape(-1))
    pk = jnp.concatenate([jnp.asarray(_DIST_T), jnp.asarray(_D_SPECIAL),
                          hdr, trm])
    out = _run(x.reshape(GS, GS), pk, (h, m, s), nb_rules, nterms)
    return (pos, out.reshape(N, 1))


# speedup vs baseline: 1.0817x; 1.0020x over previous
import functools

import numpy as np
import jax
import jax.numpy as jnp
from jax import lax
from jax.experimental import pallas as pl
from jax.experimental.pallas import tpu as pltpu
from jax.experimental.pallas import tpu_sc as plsc

GS = 128
N = GS * GS
R = 15
K = 2 * R + 1
KP = 32
NB = 1089
NSHIFT = K * K
NPAD = NB - NSHIFT
T = 10.0
L = 16
DT_LEN = K * KP
PADW = GS + 2 * R

_dt = np.full((K, KP), 1.0e4, np.float32)
for _si in range(K):
    for _sj in range(K):
        _dt[_si, _sj] = np.sqrt((_si - R) ** 2 + (_sj - R) ** 2) / R
_DIST_T = _dt.reshape(-1)
_D_SPECIAL = np.array([0.0, np.sqrt(2.0) * (GS // 2) / R] + [1.0e4] * (L - 2),
                      np.float32)


def _wterm(d, p_rk, p_b, p_w, p_r):
    z = (d / p_r - p_rk) / p_w
    return p_b * jnp.exp(-(z * z) / 2.0)


def _sc_body(pk_hbm, wn_hbm, pk_v, wt_v, *, nb_rules, nterms, npk):
    wid = lax.axis_index("s") * 2 + lax.axis_index("c")

    @pl.when(wid == 0)
    def _():
        pltpu.sync_copy(pk_hbm, pk_v)

        spec = pk_v[pl.ds(DT_LEN, L)]
        nhdr = ((4 * nb_rules + L - 1) // L) * L
        hchunks = [pk_v[pl.ds(DT_LEN + L + c * L, L)] for c in range(nhdr // L)]
        tbase = DT_LEN + L + nhdr
        ntrm = ((3 * nb_rules * nterms + L - 1) // L) * L
        tchunks = [pk_v[pl.ds(tbase + c * L, L)] for c in range(ntrm // L)]

        def term_params(kr, lt):
            base = (kr * nterms + lt) * 3
            return (tchunks[base // L][base % L],
                    tchunks[(base + 1) // L][(base + 1) % L],
                    tchunks[(base + 2) // L][(base + 2) % L])

        for kr in range(nb_rules):
            p_r = hchunks[(kr * 4) // L][(kr * 4) % L]
            sig_spec = 1.0 / (1.0 + jnp.exp((spec - 1.0) * 10.0))
            wspec = jnp.zeros((L,), jnp.float32)
            for lt in range(nterms):
                p_rk, p_b, p_w = term_params(kr, lt)
                wspec = wspec + _wterm(spec, p_rk, p_b, p_w, p_r)
            wspec = sig_spec * wspec
            wsum_vec = jnp.zeros((L,), jnp.float32)
            wvs = []
            for c in range(DT_LEN // L):
                dvec = pk_v[pl.ds(c * L, L)]
                sig = 1.0 / (1.0 + jnp.exp((dvec - 1.0) * 10.0))
                core = jnp.zeros((L,), jnp.float32)
                for lt in range(nterms):
                    p_rk, p_b, p_w = term_params(kr, lt)
                    core = core + _wterm(dvec, p_rk, p_b, p_w, p_r)
                wv = sig * core
                wvs.append(wv)
                wsum_vec = wsum_vec + wv
            w0 = wspec[0]
            wpad = wspec[1]
            wsum = w0 + NPAD * wpad
            for l in range(L):
                wsum = wsum + wsum_vec[l]
            inv_v = 1.0 / (jnp.full((L,), wsum) * float(NB + 1))
            for c in range(DT_LEN // L):
                wt_v[pl.ds(kr * DT_LEN + c * L, L)] = wvs[c] * inv_v
            inv0 = inv_v[0]
            lanes = lax.iota(jnp.int32, L)
            sl_vec = jnp.where(lanes == 0, inv0 * w0,
                               jnp.where(lanes == 1,
                                         inv0 * (float(NPAD) * wpad), 0.0))
            wt_v[pl.ds(nb_rules * DT_LEN + kr * L, L)] = sl_vec
        pltpu.sync_copy(wt_v, wn_hbm)


def _run_sc(pk, nb_rules, nterms):
    mesh = plsc.VectorSubcoreMesh(core_axis_name="c", subcore_axis_name="s",
                                  num_cores=1, num_subcores=1)
    body = functools.partial(_sc_body, nb_rules=nb_rules, nterms=nterms,
                             npk=pk.shape[0])
    f = pl.kernel(
        body,
        out_type=jax.ShapeDtypeStruct((nb_rules * (DT_LEN + L),), jnp.float32),
        mesh=mesh,
        compiler_params=pltpu.CompilerParams(needs_layout_passes=False),
        scratch_types=[
            pltpu.VMEM((pk.shape[0],), jnp.float32),
            pltpu.VMEM((nb_rules * (DT_LEN + L),), jnp.float32),
        ],
    )
    return f(pk)


def _tc_body(x_ref, wn_ref, sl_ref, h_ref, m_ref, s_ref, out_ref, *, nb_rules):
    xg = x_ref[...]

    xv = jnp.concatenate([xg[GS - R:, :], xg, xg[:R, :]], axis=0)
    xp = jnp.concatenate([xv[:, GS - R:], xv, xv[:, :R]], axis=1)

    G = jnp.concatenate([xp[si:si + GS, :] for si in range(K)], axis=1)

    a3 = jax.lax.broadcasted_iota(jnp.int32, (K, PADW, GS), 1)
    j3 = jax.lax.broadcasted_iota(jnp.int32, (K, PADW, GS), 2)
    t3 = jax.lax.broadcasted_iota(jnp.int32, (K, PADW, GS), 0)
    masks = (a3 - j3 == t3).astype(jnp.float32).reshape(K, PADW * GS)

    delta = jnp.zeros_like(xg)
    for kr in range(nb_rules):
        kwn = wn_ref[kr * K:(kr + 1) * K, :K]
        w0n = sl_ref[kr:kr + 1, 0:1]
        padn = sl_ref[kr:kr + 1, 1:2]

        B = jnp.dot(kwn, masks, preferred_element_type=jnp.float32)
        B = B.reshape(K * PADW, GS)

        conv = jnp.dot(G, B, preferred_element_type=jnp.float32)
        pot = conv + w0n * xg + padn * xg[0:1, 0:1]

        mk = m_ref[kr:kr + 1, 0:1]
        sk = s_ref[kr:kr + 1, 0:1]
        field = jnp.exp(-(pot - mk) ** 2 / (2.0 * sk * sk) - 0.001) * 2.0 - 1.0
        delta = delta + h_ref[kr:kr + 1, 0:1] * field

    out_ref[...] = jnp.clip(xg + delta * (1.0 / T), 0.0, 1.0)


def _run_tc(xg, wn2d, sl2d, h, m, s):
    nb_rules = h.shape[0]
    body = functools.partial(_tc_body, nb_rules=nb_rules)
    return pl.pallas_call(
        body,
        out_shape=jax.ShapeDtypeStruct((GS, GS), jnp.float32),
    )(xg, wn2d, sl2d, h.reshape(-1, 1), m.reshape(-1, 1), s.reshape(-1, 1))


def _pad16(a):
    pad = (-a.shape[0]) % 16
    return jnp.pad(a, (0, pad)) if pad else a


@functools.partial(jax.jit, static_argnums=(3, 4))
def _run(xg, pk, hms, nb_rules, nterms):
    h, m, s = hms
    wnsl = _run_sc(pk, nb_rules, nterms)
    wn2d = wnsl[:nb_rules * DT_LEN].reshape(nb_rules * K, KP)
    sl2d = wnsl[nb_rules * DT_LEN:].reshape(nb_rules, L)
    return _run_tc(xg, wn2d, sl2d, h, m, s)


def kernel(pos, x, r, rk, b, w, h, m, s):
    nb_rules = r.shape[0]
    nterms = rk.shape[1]
    hdr = _pad16(jnp.stack([r, h, m, s], axis=1).reshape(-1))
    trm = _pad16(jnp.stack([rk, jnp.broadcast_to(b, rk.shape),
                            jnp.broadcast_to(w, rk.shape)], axis=2).reshape(-1))
    pk = jnp.concatenate([jnp.asarray(_DIST_T), jnp.asarray(_D_SPECIAL),
                          hdr, trm])
    out = _run(x.reshape(GS, GS), pk, (h, m, s), nb_rules, nterms)
    return (pos, out.reshape(N, 1))


# speedup vs baseline: 1.0858x; 1.0037x over previous
import functools

import numpy as np
import jax
import jax.numpy as jnp
from jax import lax
from jax.experimental import pallas as pl
from jax.experimental.pallas import tpu as pltpu
from jax.experimental.pallas import tpu_sc as plsc

GS = 128
N = GS * GS
R = 15
K = 2 * R + 1
KP = 32
NB = 1089
NSHIFT = K * K
NPAD = NB - NSHIFT
T = 10.0
L = 16
DT_LEN = K * KP
PADW = GS + 2 * R

_dt = np.full((K, KP), 1.0e4, np.float32)
for _si in range(K):
    for _sj in range(K):
        _dt[_si, _sj] = np.sqrt((_si - R) ** 2 + (_sj - R) ** 2) / R
_DIST_T = _dt.reshape(-1)
_D_SPECIAL = np.array([0.0, np.sqrt(2.0) * (GS // 2) / R] + [1.0e4] * (L - 2),
                      np.float32)


def _wterm(d, p_rk, p_b, p_w, p_r):
    z = (d / p_r - p_rk) / p_w
    return p_b * jnp.exp(-(z * z) / 2.0)


def _sc_body(pk_hbm, wn_hbm, pk_v, wt_v, *, nb_rules, nterms):
    wid = lax.axis_index("s") * 2 + lax.axis_index("c")

    @pl.when(wid == 0)
    def _():
        pltpu.sync_copy(pk_hbm, pk_v)

        spec = pk_v[pl.ds(DT_LEN, L)]
        nhdr = ((4 * nb_rules + L - 1) // L) * L
        hchunks = [pk_v[pl.ds(DT_LEN + L + c * L, L)] for c in range(nhdr // L)]
        tbase = DT_LEN + L + nhdr
        ntrm = ((3 * nb_rules * nterms + L - 1) // L) * L
        tchunks = [pk_v[pl.ds(tbase + c * L, L)] for c in range(ntrm // L)]

        def term_params(kr, lt):
            base = (kr * nterms + lt) * 3
            return (tchunks[base // L][base % L],
                    tchunks[(base + 1) // L][(base + 1) % L],
                    tchunks[(base + 2) // L][(base + 2) % L])

        for kr in range(nb_rules):
            p_r = hchunks[(kr * 4) // L][(kr * 4) % L]
            sig_spec = 1.0 / (1.0 + jnp.exp((spec - 1.0) * 10.0))
            wspec = jnp.zeros((L,), jnp.float32)
            for lt in range(nterms):
                p_rk, p_b, p_w = term_params(kr, lt)
                wspec = wspec + _wterm(spec, p_rk, p_b, p_w, p_r)
            wspec = sig_spec * wspec
            wsum_vec = jnp.zeros((L,), jnp.float32)
            wvs = []
            for c in range(DT_LEN // L):
                dvec = pk_v[pl.ds(c * L, L)]
                sig = 1.0 / (1.0 + jnp.exp((dvec - 1.0) * 10.0))
                core = jnp.zeros((L,), jnp.float32)
                for lt in range(nterms):
                    p_rk, p_b, p_w = term_params(kr, lt)
                    core = core + _wterm(dvec, p_rk, p_b, p_w, p_r)
                wv = sig * core
                wvs.append(wv)
                wsum_vec = wsum_vec + wv
            w0 = wspec[0]
            wpad = wspec[1]
            wsum = w0 + NPAD * wpad
            for l in range(L):
                wsum = wsum + wsum_vec[l]
            inv_v = 1.0 / (jnp.full((L,), wsum) * float(NB + 1))
            for c in range(DT_LEN // L):
                wt_v[pl.ds(kr * DT_LEN + c * L, L)] = wvs[c] * inv_v
            inv0 = inv_v[0]
            lanes = lax.iota(jnp.int32, L)
            sl_vec = jnp.where(lanes == 0, inv0 * w0,
                               jnp.where(lanes == 1,
                                         inv0 * (float(NPAD) * wpad), 0.0))
            wt_v[pl.ds(nb_rules * DT_LEN + kr * L, L)] = sl_vec
        pltpu.sync_copy(wt_v, wn_hbm)


def _run_sc(pk, nb_rules, nterms):
    mesh = plsc.VectorSubcoreMesh(core_axis_name="c", subcore_axis_name="s",
                                  num_cores=1, num_subcores=1)
    body = functools.partial(_sc_body, nb_rules=nb_rules, nterms=nterms)
    f = pl.kernel(
        body,
        out_type=jax.ShapeDtypeStruct((nb_rules * (DT_LEN + L),), jnp.float32),
        mesh=mesh,
        compiler_params=pltpu.CompilerParams(needs_layout_passes=False),
        scratch_types=[
            pltpu.VMEM((pk.shape[0],), jnp.float32),
            pltpu.VMEM((nb_rules * (DT_LEN + L),), jnp.float32),
        ],
    )
    return f(pk)


def _tc_body(x_ref, wn_ref, sl_ref, h_ref, m_ref, s_ref, out_ref, *, nb_rules):
    xg = x_ref[...]

    xv = jnp.concatenate([xg[GS - R:, :], xg, xg[:R, :]], axis=0)
    xp = jnp.concatenate([xv[:, GS - R:], xv, xv[:, :R]], axis=1)

    G = jnp.concatenate([xp[si:si + GS, :] for si in range(K)], axis=1)

    a3 = jax.lax.broadcasted_iota(jnp.int32, (K, PADW, GS), 1)
    j3 = jax.lax.broadcasted_iota(jnp.int32, (K, PADW, GS), 2)
    t3 = jax.lax.broadcasted_iota(jnp.int32, (K, PADW, GS), 0)
    masks = (a3 - j3 == t3).astype(jnp.float32).reshape(K, PADW * GS)

    delta = jnp.zeros_like(xg)
    for kr in range(nb_rules):
        kwn = wn_ref[kr * K:(kr + 1) * K, :K]
        w0n = sl_ref[kr:kr + 1, 0:1]
        padn = sl_ref[kr:kr + 1, 1:2]

        B = jnp.dot(kwn, masks, preferred_element_type=jnp.float32)
        B = B.reshape(K * PADW, GS)

        conv = jnp.dot(G, B, preferred_element_type=jnp.float32)
        pot = conv + w0n * xg + padn * xg[0:1, 0:1]

        mk = m_ref[kr:kr + 1, 0:1]
        sk = s_ref[kr:kr + 1, 0:1]
        field = jnp.exp(-(pot - mk) ** 2 / (2.0 * sk * sk) - 0.001) * 2.0 - 1.0
        delta = delta + h_ref[kr:kr + 1, 0:1] * field

    out_ref[...] = jnp.clip(xg + delta * (1.0 / T), 0.0, 1.0)


def _run_tc(xg, wn2d, sl2d, h, m, s):
    nb_rules = h.shape[0]
    body = functools.partial(_tc_body, nb_rules=nb_rules)
    return pl.pallas_call(
        body,
        out_shape=jax.ShapeDtypeStruct((GS, GS), jnp.float32),
    )(xg, wn2d, sl2d, h.reshape(-1, 1), m.reshape(-1, 1), s.reshape(-1, 1))


def _pad16(a):
    pad = (-a.shape[0]) % 16
    return jnp.pad(a, (0, pad)) if pad else a


@functools.partial(jax.jit, static_argnums=(3, 4))
def _run(xg, pk, hms, nb_rules, nterms):
    h, m, s = hms
    wnsl = _run_sc(pk, nb_rules, nterms)
    wn2d = wnsl[:nb_rules * DT_LEN].reshape(nb_rules * K, KP)
    sl2d = wnsl[nb_rules * DT_LEN:].reshape(nb_rules, L)
    return _run_tc(xg, wn2d, sl2d, h, m, s)


def kernel(pos, x, r, rk, b, w, h, m, s):
    nb_rules = r.shape[0]
    nterms = rk.shape[1]
    hdr = _pad16(jnp.stack([r, h, m, s], axis=1).reshape(-1))
    trm = _pad16(jnp.stack([rk, jnp.broadcast_to(b, rk.shape),
                            jnp.broadcast_to(w, rk.shape)], axis=2).reshape(-1))
    pk = jnp.concatenate([jnp.asarray(_DIST_T), jnp.asarray(_D_SPECIAL),
                          hdr, trm])
    out = _run(x.reshape(GS, GS), pk, (h, m, s), nb_rules, nterms)
    return (pos, out.reshape(N, 1))


# speedup vs baseline: 1.0898x; 1.0037x over previous
import functools

import numpy as np
import jax
import jax.numpy as jnp
from jax import lax
from jax.experimental import pallas as pl
from jax.experimental.pallas import tpu as pltpu
from jax.experimental.pallas import tpu_sc as plsc

GS = 128
N = GS * GS
R = 15
K = 2 * R + 1
KP = 32
NB = 1089
NSHIFT = K * K
NPAD = NB - NSHIFT
T = 10.0
L = 16
DT_LEN = K * KP
PADW = GS + 2 * R

_dt = np.full((K, KP), 1.0e4, np.float32)
for _si in range(K):
    for _sj in range(K):
        _dt[_si, _sj] = np.sqrt((_si - R) ** 2 + (_sj - R) ** 2) / R
_DIST_T = _dt.reshape(-1)
_D_SPECIAL = np.array([0.0, np.sqrt(2.0) * (GS // 2) / R] + [1.0e4] * (L - 2),
                      np.float32)


def _wterm(d, p_rk, p_b, p_w, p_r):
    z = (d / p_r - p_rk) / p_w
    return p_b * jnp.exp(-(z * z) / 2.0)


def _sc_body(pk_hbm, wn_hbm, pk_v, wt_v, *, nb_rules, nterms):
    wid = lax.axis_index("s") * 2 + lax.axis_index("c")

    @pl.when(wid == 0)
    def _():
        pltpu.sync_copy(pk_hbm, pk_v)

        spec = pk_v[pl.ds(DT_LEN, L)]
        nhdr = ((4 * nb_rules + L - 1) // L) * L
        hchunks = [pk_v[pl.ds(DT_LEN + L + c * L, L)] for c in range(nhdr // L)]
        tbase = DT_LEN + L + nhdr
        ntrm = ((3 * nb_rules * nterms + L - 1) // L) * L
        tchunks = [pk_v[pl.ds(tbase + c * L, L)] for c in range(ntrm // L)]

        def term_params(kr, lt):
            base = (kr * nterms + lt) * 3
            return (tchunks[base // L][base % L],
                    tchunks[(base + 1) // L][(base + 1) % L],
                    tchunks[(base + 2) // L][(base + 2) % L])

        for kr in range(nb_rules):
            p_r = hchunks[(kr * 4) // L][(kr * 4) % L]
            sig_spec = 1.0 / (1.0 + jnp.exp((spec - 1.0) * 10.0))
            wspec = jnp.zeros((L,), jnp.float32)
            for lt in range(nterms):
                p_rk, p_b, p_w = term_params(kr, lt)
                wspec = wspec + _wterm(spec, p_rk, p_b, p_w, p_r)
            wspec = sig_spec * wspec
            wsum_vec = jnp.zeros((L,), jnp.float32)
            wvs = []
            for c in range(DT_LEN // L):
                dvec = pk_v[pl.ds(c * L, L)]
                sig = 1.0 / (1.0 + jnp.exp((dvec - 1.0) * 10.0))
                core = jnp.zeros((L,), jnp.float32)
                for lt in range(nterms):
                    p_rk, p_b, p_w = term_params(kr, lt)
                    core = core + _wterm(dvec, p_rk, p_b, p_w, p_r)
                wv = sig * core
                wvs.append(wv)
                wsum_vec = wsum_vec + wv
            w0 = wspec[0]
            wpad = wspec[1]
            wsum = w0 + NPAD * wpad
            for l in range(L):
                wsum = wsum + wsum_vec[l]
            inv_v = 1.0 / (jnp.full((L,), wsum) * float(NB + 1))
            for c in range(DT_LEN // L):
                wt_v[pl.ds(kr * DT_LEN + c * L, L)] = wvs[c] * inv_v
            inv0 = inv_v[0]
            lanes = lax.iota(jnp.int32, L)
            sl_vec = jnp.where(lanes == 0, inv0 * w0,
                               jnp.where(lanes == 1,
                                         inv0 * (float(NPAD) * wpad), 0.0))
            wt_v[pl.ds(nb_rules * DT_LEN + kr * L, L)] = sl_vec
        pltpu.sync_copy(wt_v, wn_hbm)


def _run_sc(pk, nb_rules, nterms):
    mesh = plsc.VectorSubcoreMesh(core_axis_name="c", subcore_axis_name="s",
                                  num_cores=1, num_subcores=1)
    body = functools.partial(_sc_body, nb_rules=nb_rules, nterms=nterms)
    f = pl.kernel(
        body,
        out_type=jax.ShapeDtypeStruct((nb_rules * (DT_LEN + L),), jnp.float32),
        mesh=mesh,
        compiler_params=pltpu.CompilerParams(needs_layout_passes=False),
        scratch_types=[
            pltpu.VMEM((pk.shape[0],), jnp.float32),
            pltpu.VMEM((nb_rules * (DT_LEN + L),), jnp.float32),
        ],
    )
    return f(pk)


def _tc_body(x_ref, wn_ref, sl_ref, h_ref, m_ref, s_ref, out_ref, *, nb_rules):
    xg = x_ref[...]

    xv = jnp.concatenate([xg[GS - R:, :], xg, xg[:R, :]], axis=0)
    xp = jnp.concatenate([xv[:, GS - R:], xv, xv[:, :R]], axis=1)

    G = jnp.concatenate([xp[si:si + GS, :] for si in range(K)], axis=1)

    t2 = jax.lax.broadcasted_iota(jnp.int32, (K, PADW * GS), 0)
    u2 = jax.lax.broadcasted_iota(jnp.int32, (K, PADW * GS), 1)
    masks = ((u2 >> 7) - (u2 & (GS - 1)) == t2).astype(jnp.float32)

    delta = jnp.zeros_like(xg)
    for kr in range(nb_rules):
        kwn = wn_ref[kr * K:(kr + 1) * K, :K]
        w0n = sl_ref[kr:kr + 1, 0:1]
        padn = sl_ref[kr:kr + 1, 1:2]

        B = jnp.dot(kwn, masks, preferred_element_type=jnp.float32)
        B = B.reshape(K * PADW, GS)

        conv = jnp.dot(G, B, preferred_element_type=jnp.float32)
        pot = conv + w0n * xg + padn * xg[0:1, 0:1]

        mk = m_ref[kr:kr + 1, 0:1]
        sk = s_ref[kr:kr + 1, 0:1]
        field = jnp.exp(-(pot - mk) ** 2 / (2.0 * sk * sk) - 0.001) * 2.0 - 1.0
        delta = delta + h_ref[kr:kr + 1, 0:1] * field

    out_ref[...] = jnp.clip(xg + delta * (1.0 / T), 0.0, 1.0)


def _run_tc(xg, wn2d, sl2d, h, m, s):
    nb_rules = h.shape[0]
    body = functools.partial(_tc_body, nb_rules=nb_rules)
    return pl.pallas_call(
        body,
        out_shape=jax.ShapeDtypeStruct((GS, GS), jnp.float32),
    )(xg, wn2d, sl2d, h.reshape(-1, 1), m.reshape(-1, 1), s.reshape(-1, 1))


def _pad16(a):
    pad = (-a.shape[0]) % 16
    return jnp.pad(a, (0, pad)) if pad else a


@functools.partial(jax.jit, static_argnums=(3, 4))
def _run(xg, pk, hms, nb_rules, nterms):
    h, m, s = hms
    wnsl = _run_sc(pk, nb_rules, nterms)
    wn2d = wnsl[:nb_rules * DT_LEN].reshape(nb_rules * K, KP)
    sl2d = wnsl[nb_rules * DT_LEN:].reshape(nb_rules, L)
    return _run_tc(xg, wn2d, sl2d, h, m, s)


def kernel(pos, x, r, rk, b, w, h, m, s):
    nb_rules = r.shape[0]
    nterms = rk.shape[1]
    hdr = _pad16(jnp.stack([r, h, m, s], axis=1).reshape(-1))
    trm = _pad16(jnp.stack([rk, jnp.broadcast_to(b, rk.shape),
                            jnp.broadcast_to(w, rk.shape)], axis=2).reshape(-1))
    pk = jnp.concatenate([jnp.asarray(_DIST_T), jnp.asarray(_D_SPECIAL),
                          hdr, trm])
    out = _run(x.reshape(GS, GS), pk, (h, m, s), nb_rules, nterms)
    return (pos, out.reshape(N, 1))
